# baseline (speedup 1.0000x reference)
import functools

import jax
import jax.numpy as jnp
from jax import lax
from jax.experimental import pallas as pl

_TH = 16


def _body(x_ref, xh_ref, w1_ref, b1_ref, wd_ref, bd_ref, w2_ref,
          b2_ref, out_ref, *, th, w, hidden, c_in, c_out):
    i = pl.program_id(1)
    nh = pl.num_programs(1)
    s = th * w

    x2d = x_ref[0].reshape(c_in, s)
    h = jnp.dot(w1_ref[...], x2d, preferred_element_type=jnp.float32)
    h = h + b1_ref[...]
    xg = h[:hidden]
    v = h[hidden:]

    hb = jnp.dot(w1_ref[:hidden], xh_ref[0, 0],
                 preferred_element_type=jnp.float32) + b1_ref[:hidden]
    top = hb[:, :w] * (i != 0).astype(jnp.float32)
    bot = hb[:, w:] * (i != nh - 1).astype(jnp.float32)

    up = jnp.concatenate([top, xg[:, :s - w]], axis=1)
    dn = jnp.concatenate([xg[:, w:], bot], axis=1)

    lane = lax.broadcasted_iota(jnp.int32, (1, s), 1)
    mask_l = ((lane & (w - 1)) != 0).astype(jnp.float32)
    mask_r = ((lane & (w - 1)) != w - 1).astype(jnp.float32)
    zcol = jnp.zeros((hidden, 1), dtype=jnp.float32)

    def sr(a):
        return jnp.concatenate([zcol, a[:, :-1]], axis=1)

    def sl(a):
        return jnp.concatenate([a[:, 1:], zcol], axis=1)

    def wtap(k):
        return wd_ref[:, k].reshape(hidden, 1)

    rows = (up, xg, dn)
    left = sum(wtap(3 * dy + 0) * rows[dy] for dy in range(3))
    mid = sum(wtap(3 * dy + 1) * rows[dy] for dy in range(3))
    right = sum(wtap(3 * dy + 2) * rows[dy] for dy in range(3))
    d = bd_ref[...] + mid + mask_l * sr(left) + mask_r * sl(right)

    act = 0.5 * d * (1.0 + lax.erf(d * 0.7071067811865476))
    g = act * v

    out = jnp.dot(w2_ref[...], g, preferred_element_type=jnp.float32)
    out = out + b2_ref[...] + x2d
    out_ref[0] = out.reshape(c_out, th, w)


@jax.jit
def kernel(x, w1, b1, wd, bd, w2, b2):
    n, c_in, hgt, wid = x.shape
    two_hidden = w1.shape[0]
    hidden = two_hidden // 2
    c_out = w2.shape[0]
    th = _TH
    n_tiles = hgt // th

    top_idx = jnp.maximum(jnp.arange(n_tiles) * th - 1, 0)
    bot_idx = jnp.minimum((jnp.arange(n_tiles) + 1) * th, hgt - 1)
    halo_idx = jnp.stack([top_idx, bot_idx], axis=1).reshape(-1)
    xh = x[:, :, halo_idx, :].reshape(n, c_in, n_tiles, 2 * wid).transpose(
        0, 2, 1, 3)

    w1_2d = w1.reshape(two_hidden, c_in)
    w2_2d = w2.reshape(c_out, hidden)
    wd_2d = wd.reshape(hidden, 9)
    b1_c = b1.reshape(two_hidden, 1)
    bd_c = bd.reshape(hidden, 1)
    b2_c = b2.reshape(c_out, 1)

    body = functools.partial(_body, th=th, w=wid, hidden=hidden,
                             c_in=c_in, c_out=c_out)
    grid = (n, n_tiles)
    full = lambda idx_n, idx_i: (0, 0)
    return pl.pallas_call(
        body,
        grid=grid,
        in_specs=[
            pl.BlockSpec((1, c_in, th, wid), lambda b, i: (b, 0, i, 0)),
            pl.BlockSpec((1, 1, c_in, 2 * wid), lambda b, i: (b, i, 0, 0)),
            pl.BlockSpec((two_hidden, c_in), full),
            pl.BlockSpec((two_hidden, 1), full),
            pl.BlockSpec((hidden, 9), full),
            pl.BlockSpec((hidden, 1), full),
            pl.BlockSpec((c_out, hidden), full),
            pl.BlockSpec((c_out, 1), full),
        ],
        out_specs=pl.BlockSpec((1, c_out, th, wid), lambda b, i: (b, 0, i, 0)),
        out_shape=jax.ShapeDtypeStruct((n, c_out, hgt, wid), jnp.float32),
    )(x, xh, w1_2d, b1_c, wd_2d, bd_c, w2_2d, b2_c)


# speedup vs baseline: 1.0834x; 1.0834x over previous
import functools

import jax
import jax.numpy as jnp
from jax import lax
from jax.experimental import pallas as pl
from jax.experimental.pallas import tpu as pltpu

_TH = 32


def _body(x_ref, xh_ref, w1_ref, b1_ref, wd_ref, bd_ref, w2_ref,
          b2_ref, out_ref, *, th, w, hidden, c_in, c_out):
    i = pl.program_id(1)
    nh = pl.num_programs(1)
    s = th * w

    x2d = x_ref[0].reshape(c_in, s)
    h = jnp.dot(w1_ref[...], x2d, preferred_element_type=jnp.float32)
    h = h + b1_ref[...]
    xg = h[:hidden]
    v = h[hidden:]

    hb = jnp.dot(w1_ref[:hidden], xh_ref[0, 0],
                 preferred_element_type=jnp.float32) + b1_ref[:hidden]
    top = hb[:, :w] * (i != 0).astype(jnp.float32)
    bot = hb[:, w:] * (i != nh - 1).astype(jnp.float32)

    up = jnp.concatenate([top, xg[:, :s - w]], axis=1)
    dn = jnp.concatenate([xg[:, w:], bot], axis=1)

    lane = lax.broadcasted_iota(jnp.int32, (1, s), 1)
    mask_l = ((lane & (w - 1)) != 0).astype(jnp.float32)
    mask_r = ((lane & (w - 1)) != w - 1).astype(jnp.float32)
    zcol = jnp.zeros((hidden, 1), dtype=jnp.float32)

    def sr(a):
        return jnp.concatenate([zcol, a[:, :-1]], axis=1)

    def sl(a):
        return jnp.concatenate([a[:, 1:], zcol], axis=1)

    def wtap(k):
        return wd_ref[:, k].reshape(hidden, 1)

    rows = (up, xg, dn)
    left = sum(wtap(3 * dy + 0) * rows[dy] for dy in range(3))
    mid = sum(wtap(3 * dy + 1) * rows[dy] for dy in range(3))
    right = sum(wtap(3 * dy + 2) * rows[dy] for dy in range(3))
    d = bd_ref[...] + mid + mask_l * sr(left) + mask_r * sl(right)

    act = 0.5 * d * (1.0 + lax.erf(d * 0.7071067811865476))
    g = act * v

    out = jnp.dot(w2_ref[...], g, preferred_element_type=jnp.float32)
    out = out + b2_ref[...] + x2d
    out_ref[0] = out.reshape(c_out, th, w)


@jax.jit
def kernel(x, w1, b1, wd, bd, w2, b2):
    n, c_in, hgt, wid = x.shape
    two_hidden = w1.shape[0]
    hidden = two_hidden // 2
    c_out = w2.shape[0]
    th = _TH
    n_tiles = hgt // th

    top_idx = jnp.maximum(jnp.arange(n_tiles) * th - 1, 0)
    bot_idx = jnp.minimum((jnp.arange(n_tiles) + 1) * th, hgt - 1)
    halo_idx = jnp.stack([top_idx, bot_idx], axis=1).reshape(-1)
    xh = x[:, :, halo_idx, :].reshape(n, c_in, n_tiles, 2 * wid).transpose(
        0, 2, 1, 3)

    w1_2d = w1.reshape(two_hidden, c_in)
    w2_2d = w2.reshape(c_out, hidden)
    wd_2d = wd.reshape(hidden, 9)
    b1_c = b1.reshape(two_hidden, 1)
    bd_c = bd.reshape(hidden, 1)
    b2_c = b2.reshape(c_out, 1)

    body = functools.partial(_body, th=th, w=wid, hidden=hidden,
                             c_in=c_in, c_out=c_out)
    grid = (n, n_tiles)
    full = lambda idx_n, idx_i: (0, 0)
    return pl.pallas_call(
        body,
        grid=grid,
        in_specs=[
            pl.BlockSpec((1, c_in, th, wid), lambda b, i: (b, 0, i, 0)),
            pl.BlockSpec((1, 1, c_in, 2 * wid), lambda b, i: (b, i, 0, 0)),
            pl.BlockSpec((two_hidden, c_in), full),
            pl.BlockSpec((two_hidden, 1), full),
            pl.BlockSpec((hidden, 9), full),
            pl.BlockSpec((hidden, 1), full),
            pl.BlockSpec((c_out, hidden), full),
            pl.BlockSpec((c_out, 1), full),
        ],
        out_specs=pl.BlockSpec((1, c_out, th, wid), lambda b, i: (b, 0, i, 0)),
        out_shape=jax.ShapeDtypeStruct((n, c_out, hgt, wid), jnp.float32),
        compiler_params=pltpu.CompilerParams(
            dimension_semantics=("parallel", "parallel")),
    )(x, xh, w1_2d, b1_c, wd_2d, bd_c, w2_2d, b2_c)


# speedup vs baseline: 1.3016x; 1.2014x over previous
import functools

import jax
import jax.numpy as jnp
from jax import lax
from jax.experimental import pallas as pl
from jax.experimental.pallas import tpu as pltpu

_TH = 32


def _body(x_ref, xh_ref, w1_ref, b1_ref, wd_ref, bd_ref, w2_ref,
          b2_ref, out_ref, *, th, w, hidden, c_in, c_out):
    i = pl.program_id(1)
    nh = pl.num_programs(1)
    s = th * w

    bf = jnp.bfloat16
    x2d = x_ref[0].astype(bf).reshape(c_in, s)
    w1b = w1_ref[...].astype(bf)
    b1b = b1_ref[...].astype(bf)
    h = jnp.dot(w1b, x2d, preferred_element_type=jnp.float32)
    h = (h + b1b.astype(jnp.float32)).astype(bf)
    xg = h[:hidden]
    v = h[hidden:]

    hb = (jnp.dot(w1b[:hidden], xh_ref[0, 0].astype(bf),
                  preferred_element_type=jnp.float32)
          + b1_ref[:hidden]).astype(bf)
    top = hb[:, :w] * (i != 0).astype(bf)
    bot = hb[:, w:] * (i != nh - 1).astype(bf)

    up = jnp.concatenate([top, xg[:, :s - w]], axis=1)
    dn = jnp.concatenate([xg[:, w:], bot], axis=1)

    lane = lax.broadcasted_iota(jnp.int32, (1, s), 1)
    mask_l = ((lane & (w - 1)) != 0).astype(bf)
    mask_r = ((lane & (w - 1)) != w - 1).astype(bf)
    zcol = jnp.zeros((hidden, 1), dtype=bf)

    def sr(a):
        return jnp.concatenate([zcol, a[:, :-1]], axis=1)

    def sl(a):
        return jnp.concatenate([a[:, 1:], zcol], axis=1)

    def wtap(k):
        return wd_ref[:, k].reshape(hidden, 1).astype(bf)

    rows = (up, xg, dn)
    left = sum(wtap(3 * dy + 0) * rows[dy] for dy in range(3))
    mid = sum(wtap(3 * dy + 1) * rows[dy] for dy in range(3))
    right = sum(wtap(3 * dy + 2) * rows[dy] for dy in range(3))
    d = (mid + mask_l * sr(left) + mask_r * sl(right)).astype(
        jnp.float32) + bd_ref[...]

    act = 0.5 * d * (1.0 + lax.erf(d * 0.7071067811865476))
    g = act.astype(bf) * v

    out = jnp.dot(w2_ref[...].astype(bf), g,
                  preferred_element_type=jnp.float32)
    out = (out + b2_ref[...]).reshape(c_out, th, w)
    out_ref[0] = out + x_ref[0]


@jax.jit
def kernel(x, w1, b1, wd, bd, w2, b2):
    n, c_in, hgt, wid = x.shape
    two_hidden = w1.shape[0]
    hidden = two_hidden // 2
    c_out = w2.shape[0]
    th = _TH
    n_tiles = hgt // th

    top_idx = jnp.maximum(jnp.arange(n_tiles) * th - 1, 0)
    bot_idx = jnp.minimum((jnp.arange(n_tiles) + 1) * th, hgt - 1)
    halo_idx = jnp.stack([top_idx, bot_idx], axis=1).reshape(-1)
    xh = x[:, :, halo_idx, :].reshape(n, c_in, n_tiles, 2 * wid).transpose(
        0, 2, 1, 3)

    w1_2d = w1.reshape(two_hidden, c_in)
    w2_2d = w2.reshape(c_out, hidden)
    wd_2d = wd.reshape(hidden, 9)
    b1_c = b1.reshape(two_hidden, 1)
    bd_c = bd.reshape(hidden, 1)
    b2_c = b2.reshape(c_out, 1)

    body = functools.partial(_body, th=th, w=wid, hidden=hidden,
                             c_in=c_in, c_out=c_out)
    grid = (n, n_tiles)
    full = lambda idx_n, idx_i: (0, 0)
    return pl.pallas_call(
        body,
        grid=grid,
        in_specs=[
            pl.BlockSpec((1, c_in, th, wid), lambda b, i: (b, 0, i, 0)),
            pl.BlockSpec((1, 1, c_in, 2 * wid), lambda b, i: (b, i, 0, 0)),
            pl.BlockSpec((two_hidden, c_in), full),
            pl.BlockSpec((two_hidden, 1), full),
            pl.BlockSpec((hidden, 9), full),
            pl.BlockSpec((hidden, 1), full),
            pl.BlockSpec((c_out, hidden), full),
            pl.BlockSpec((c_out, 1), full),
        ],
        out_specs=pl.BlockSpec((1, c_out, th, wid), lambda b, i: (b, 0, i, 0)),
        out_shape=jax.ShapeDtypeStruct((n, c_out, hgt, wid), jnp.float32),
        compiler_params=pltpu.CompilerParams(
            dimension_semantics=("parallel", "parallel")),
    )(x, xh, w1_2d, b1_c, wd_2d, bd_c, w2_2d, b2_c)
